# Initial kernel scaffold; baseline (speedup 1.0000x reference)
import jax
import jax.numpy as jnp
from jax.experimental import pallas as pl


def kernel(features, positions, edge_index, W1, b1, W2, b2, W3, b3):
    raise NotImplementedError("write your pallas kernel here")


# speedup vs baseline: 3.0866x; 3.0866x over previous
import functools
import math

import jax
import jax.numpy as jnp
from jax import lax
from jax.experimental import pallas as pl
from jax.experimental.pallas import tpu as pltpu
from jax.experimental.pallas import tpu_sc as plsc

NC, NS = 2, 16
NW = NC * NS
CHUNK = 128
BN = 1000
BE = 2048


def _prep_body(n_sh, c_out, feat, pos, w1n, b1, w3, b3, t_out, p_out, w3e_out,
               b3e_out):
    f1 = jnp.dot(feat[...], w1n[...], preferred_element_type=jnp.float32)
    f1 = f1 + b1[...]
    rows = feat.shape[0]
    h_dim = f1.shape[1]
    posv = pos[...]
    t_out[...] = jnp.concatenate(
        [f1, posv, jnp.zeros((rows, 128 - h_dim - 3), jnp.float32)], axis=1)
    p_out[...] = jnp.concatenate(
        [posv, jnp.zeros((rows, 125), jnp.float32)], axis=1)
    w3v = w3[...]
    w3e = w3v[:, 0:c_out]
    for k in range(1, n_sh):
        w3e = w3e + w3v[:, k * c_out:(k + 1) * c_out]
    w3e_out[...] = w3e
    b3v = b3[...]
    b3e = b3v[:, 0:c_out]
    for k in range(1, n_sh):
        b3e = b3e + b3v[:, k * c_out:(k + 1) * c_out]
    b3e_out[...] = b3e


def _gather_body(nch, h_dim, rowg, colg, p_hbm, t_hbm, r1_out, r2_out, idr,
                 idc, bufp, buft, sem1, sem2):
    cid = lax.axis_index("c")
    sid = lax.axis_index("s")
    wid = sid * NC + cid
    pltpu.sync_copy(rowg.at[pl.ds(wid * nch, nch)], idr)
    pltpu.sync_copy(colg.at[pl.ds(wid * nch, nch)], idc)

    def chunk(j, carry):
        base = (wid * nch + j) * CHUNK
        cp1 = pltpu.async_copy(p_hbm.at[idr.at[j]], bufp, sem1)
        cp2 = pltpu.async_copy(t_hbm.at[idc.at[j]], buft, sem2)
        cp1.wait()
        cp2.wait()
        pltpu.sync_copy(bufp, r1_out.at[pl.ds(base, CHUNK)])
        pltpu.sync_copy(buft, r2_out.at[pl.ds(base, CHUNK)])
        return carry

    lax.fori_loop(0, nch, chunk, 0)


def _mlp_body(h_dim, r1, r2, w1s, w2, b2, w3e, b3e, out):
    r1v = r1[...]
    r2v = r2[...]
    dx = r1v[:, 0:1] - r2v[:, h_dim + 0:h_dim + 1]
    dy = r1v[:, 1:2] - r2v[:, h_dim + 1:h_dim + 2]
    dz = r1v[:, 2:3] - r2v[:, h_dim + 2:h_dim + 3]
    n2 = dx * dx + dy * dy + dz * dz
    inv = 1.0 / jnp.maximum(jnp.sqrt(n2), 1e-8)
    x = dx * inv
    y = dy * inv
    z = dz * inv
    w1sv = w1s[...]
    c0 = 0.5 / math.sqrt(math.pi)
    cx = 0.5 * math.sqrt(3.0 / (2.0 * math.pi))
    cz = 0.5 * math.sqrt(3.0 / math.pi)
    cxy = 0.25 * math.sqrt(15.0 / (2.0 * math.pi))
    czx = 0.5 * math.sqrt(15.0 / (2.0 * math.pi))
    czz = 0.25 * math.sqrt(5.0 / math.pi)
    u0 = c0 * w1sv[0:1, :]
    u1 = cx * (w1sv[3:4, :] - w1sv[1:2, :])
    u2 = cz * w1sv[2:3, :]
    u3 = cxy * (w1sv[4:5, :] + w1sv[8:9, :])
    u4 = czx * (w1sv[7:8, :] - w1sv[5:6, :])
    u5 = czz * w1sv[6:7, :]
    p1 = x * x - y * y
    p2 = z * x
    p3 = 2.0 * z * z - x * x - y * y
    h = r2v[:, 0:h_dim] + u0 + x * u1 + z * u2 + p1 * u3 + p2 * u4 + p3 * u5
    h = h * jax.nn.sigmoid(h)
    h = jnp.dot(h, w2[...], preferred_element_type=jnp.float32) + b2[...]
    h = h * jax.nn.sigmoid(h)
    out[...] = jnp.dot(h, w3e[...], preferred_element_type=jnp.float32) \
        + b3e[...]


def _scatter_body(nch, rt, rows_hbm, msg_hbm, zeros_hbm, part_out, idr, mbuf,
                  accum, sem):
    cid = lax.axis_index("c")
    sid = lax.axis_index("s")
    wid = sid * NC + cid
    pltpu.sync_copy(zeros_hbm.at[pl.ds(sid * rt, rt)],
                    accum.at[pl.ds(sid * rt, rt)])
    pltpu.sync_copy(rows_hbm.at[pl.ds(wid * nch, nch)], idr)
    plsc.subcore_barrier()

    def chunk(j, carry):
        base = (wid * nch + j) * CHUNK
        pltpu.sync_copy(msg_hbm.at[pl.ds(base, CHUNK)], mbuf)
        pltpu.sync_copy(mbuf, accum.at[idr.at[j]], add=True)
        return carry

    lax.fori_loop(0, nch, chunk, 0)
    plsc.subcore_barrier()
    pltpu.sync_copy(accum.at[pl.ds(sid * rt, rt)],
                    part_out.at[cid, pl.ds(sid * rt, rt)])


def _combine_body(part, out):
    pv = part[...]
    out[...] = pv[0] + pv[1]


def kernel(features, positions, edge_index, W1, b1, W2, b2, W3, b3):
    n, c_in = features.shape
    e = edge_index.shape[1]
    n_sh = 9
    h_dim = W1.shape[1]
    c_out = W3.shape[1] // n_sh
    f32 = jnp.float32

    ei = edge_index.astype(jnp.int32)
    row = ei[0]
    col = ei[1]
    epad = -(-e // (NW * CHUNK)) * (NW * CHUNK)
    nch = epad // (NW * CHUNK)
    pad = epad - e
    rowg = jnp.concatenate([row, jnp.zeros((pad,), jnp.int32)])
    rows = jnp.concatenate([row, jnp.full((pad,), n, jnp.int32)])
    colg = jnp.concatenate([col, jnp.zeros((pad,), jnp.int32)])
    rowg = rowg.reshape(-1, CHUNK)
    rows = rows.reshape(-1, CHUNK)
    colg = colg.reshape(-1, CHUNK)

    npad = -(-(n + 1) // (NS * 8)) * (NS * 8)
    rt = npad // NS

    w1n = W1[n_sh:]
    w1s = jnp.concatenate([W1[:n_sh], jnp.zeros((16 - n_sh, h_dim), f32)])
    b1r = b1.reshape(1, h_dim)
    b2r = b2.reshape(1, h_dim)
    b3r = b3.reshape(1, n_sh * c_out)

    nb = n // BN
    t_tab, p_tab, w3e, b3e = pl.pallas_call(
        functools.partial(_prep_body, n_sh, c_out),
        grid=(nb,),
        in_specs=[
            pl.BlockSpec((BN, c_in), lambda i: (i, 0)),
            pl.BlockSpec((BN, 3), lambda i: (i, 0)),
            pl.BlockSpec((c_in, h_dim), lambda i: (0, 0)),
            pl.BlockSpec((1, h_dim), lambda i: (0, 0)),
            pl.BlockSpec((h_dim, n_sh * c_out), lambda i: (0, 0)),
            pl.BlockSpec((1, n_sh * c_out), lambda i: (0, 0)),
        ],
        out_specs=[
            pl.BlockSpec((BN, 128), lambda i: (i, 0)),
            pl.BlockSpec((BN, 128), lambda i: (i, 0)),
            pl.BlockSpec((h_dim, c_out), lambda i: (0, 0)),
            pl.BlockSpec((1, c_out), lambda i: (0, 0)),
        ],
        out_shape=[
            jax.ShapeDtypeStruct((n, 128), f32),
            jax.ShapeDtypeStruct((n, 128), f32),
            jax.ShapeDtypeStruct((h_dim, c_out), f32),
            jax.ShapeDtypeStruct((1, c_out), f32),
        ],
    )(features, positions, w1n, b1r, W3, b3r)

    mesh = plsc.VectorSubcoreMesh(core_axis_name="c", subcore_axis_name="s")
    r1, r2 = pl.kernel(
        functools.partial(_gather_body, nch, h_dim),
        out_type=[
            jax.ShapeDtypeStruct((epad, 128), f32),
            jax.ShapeDtypeStruct((epad, 128), f32),
        ],
        mesh=mesh,
        scratch_types=[
            pltpu.VMEM((nch, CHUNK), jnp.int32),
            pltpu.VMEM((nch, CHUNK), jnp.int32),
            pltpu.VMEM((CHUNK, 128), f32),
            pltpu.VMEM((CHUNK, 128), f32),
            pltpu.SemaphoreType.DMA,
            pltpu.SemaphoreType.DMA,
        ],
    )(rowg, colg, p_tab, t_tab)

    msg = pl.pallas_call(
        functools.partial(_mlp_body, h_dim),
        grid=(epad // BE,),
        in_specs=[
            pl.BlockSpec((BE, 128), lambda i: (i, 0)),
            pl.BlockSpec((BE, 128), lambda i: (i, 0)),
            pl.BlockSpec((16, h_dim), lambda i: (0, 0)),
            pl.BlockSpec((h_dim, h_dim), lambda i: (0, 0)),
            pl.BlockSpec((1, h_dim), lambda i: (0, 0)),
            pl.BlockSpec((h_dim, c_out), lambda i: (0, 0)),
            pl.BlockSpec((1, c_out), lambda i: (0, 0)),
        ],
        out_specs=pl.BlockSpec((BE, c_out), lambda i: (i, 0)),
        out_shape=jax.ShapeDtypeStruct((epad, c_out), f32),
    )(r1, r2, w1s, W2, b2r, w3e, b3e)

    zeros_init = jnp.zeros((npad, c_out), f32)
    part = pl.kernel(
        functools.partial(_scatter_body, nch, rt),
        out_type=jax.ShapeDtypeStruct((NC, npad, c_out), f32),
        mesh=mesh,
        scratch_types=[
            pltpu.VMEM((nch, CHUNK), jnp.int32),
            pltpu.VMEM((CHUNK, c_out), f32),
            pltpu.VMEM_SHARED((npad, c_out), f32),
            pltpu.SemaphoreType.DMA,
        ],
    )(rows, msg, zeros_init)

    out = pl.pallas_call(
        _combine_body,
        grid=(nb,),
        in_specs=[pl.BlockSpec((NC, BN, c_out), lambda i: (0, i, 0))],
        out_specs=pl.BlockSpec((BN, c_out), lambda i: (i, 0)),
        out_shape=jax.ShapeDtypeStruct((n, c_out), f32),
    )(part)
    return out


# speedup vs baseline: 4.6025x; 1.4911x over previous
import functools
import math

import jax
import jax.numpy as jnp
from jax import lax
from jax.experimental import pallas as pl
from jax.experimental.pallas import tpu as pltpu
from jax.experimental.pallas import tpu_sc as plsc

NC, NS = 2, 16
NW = NC * NS
LANES = 16
CHUNK = 128
BN = 1000
BE = 2048


def _prep_body(n_sh, c_out, feat, w1n, b1, w3, b3, u_out, w3e_out, b3e_out):
    f1 = jnp.dot(feat[...], w1n[...], preferred_element_type=jnp.float32)
    f1 = f1 + b1[...]
    rows = feat.shape[0]
    h_dim = f1.shape[1]
    u_out[...] = jnp.concatenate(
        [f1, jnp.zeros((rows, 128 - h_dim), jnp.float32)], axis=1)
    w3v = w3[...]
    w3e = w3v[:, 0:c_out]
    for k in range(1, n_sh):
        w3e = w3e + w3v[:, k * c_out:(k + 1) * c_out]
    w3e_out[...] = w3e
    b3v = b3[...]
    b3e = b3v[:, 0:c_out]
    for k in range(1, n_sh):
        b3e = b3e + b3v[:, k * c_out:(k + 1) * c_out]
    b3e_out[...] = b3e


def _gather_body(nch, h_dim, rowg, colg, u_hbm, px_hbm, py_hbm, pz_hbm,
                 r2_out, ss_out, idjr, idjc, pr, pc, bufu, bufs, semu,
                 semp):
    cid = lax.axis_index("c")
    sid = lax.axis_index("s")
    wid = sid * NC + cid

    def chunk(j, carry):
        base = (wid * nch + j) * CHUNK
        pltpu.sync_copy(rowg.at[wid * nch + j], idjr)
        pltpu.sync_copy(colg.at[wid * nch + j], idjc)
        cp = pltpu.async_copy(u_hbm.at[idjc], bufu, semu)
        g0 = pltpu.async_copy(px_hbm.at[idjr], pr.at[0], semp)
        g1 = pltpu.async_copy(py_hbm.at[idjr], pr.at[1], semp)
        g2 = pltpu.async_copy(pz_hbm.at[idjr], pr.at[2], semp)
        g3 = pltpu.async_copy(px_hbm.at[idjc], pc.at[0], semp)
        g4 = pltpu.async_copy(py_hbm.at[idjc], pc.at[1], semp)
        g5 = pltpu.async_copy(pz_hbm.at[idjc], pc.at[2], semp)
        g0.wait()
        g1.wait()
        g2.wait()
        g3.wait()
        g4.wait()
        g5.wait()
        for k in range(CHUNK // LANES):
            sl = pl.ds(k * LANES, LANES)
            bufs[0, sl] = pr[0, sl] - pc[0, sl]
            bufs[1, sl] = pr[1, sl] - pc[1, sl]
            bufs[2, sl] = pr[2, sl] - pc[2, sl]
        cp.wait()
        pltpu.sync_copy(bufu, r2_out.at[pl.ds(base, CHUNK)])
        pltpu.sync_copy(bufs, ss_out.at[:, pl.ds(base, CHUNK)])
        return carry

    lax.fori_loop(0, nch, chunk, 0)


def _mlp_body(h_dim, r2, ss, w1s, w2, b2, w3e, b3e, out):
    f1 = r2[...][:, 0:h_dim]
    ssv = ss[...]
    dx = ssv[0:1, :]
    dy = ssv[1:2, :]
    dz = ssv[2:3, :]
    n2 = dx * dx + dy * dy + dz * dz
    inv = lax.rsqrt(jnp.maximum(n2, 1e-16))
    x = dx * inv
    y = dy * inv
    z = dz * inv
    p1 = x * x - y * y
    p2 = z * x
    p3 = 2.0 * z * z - x * x - y * y
    ones = jnp.ones_like(x)
    st = jnp.concatenate([x, z, p1, p2, p3, ones], axis=0)
    w1sv = w1s[...]
    c0 = 0.5 / math.sqrt(math.pi)
    cx = 0.5 * math.sqrt(3.0 / (2.0 * math.pi))
    cz = 0.5 * math.sqrt(3.0 / math.pi)
    cxy = 0.25 * math.sqrt(15.0 / (2.0 * math.pi))
    czx = 0.5 * math.sqrt(15.0 / (2.0 * math.pi))
    czz = 0.25 * math.sqrt(5.0 / math.pi)
    us = jnp.concatenate([
        cx * (w1sv[3:4, :] - w1sv[1:2, :]),
        cz * w1sv[2:3, :],
        cxy * (w1sv[4:5, :] + w1sv[8:9, :]),
        czx * (w1sv[7:8, :] - w1sv[5:6, :]),
        czz * w1sv[6:7, :],
        c0 * w1sv[0:1, :],
    ], axis=0)
    hsh = lax.dot_general(st, us, (((0,), (0,)), ((), ())),
                          preferred_element_type=jnp.float32)
    h = f1 + hsh
    h = h * jax.nn.sigmoid(h)
    h = jnp.dot(h, w2[...], preferred_element_type=jnp.float32) + b2[...]
    h = h * jax.nn.sigmoid(h)
    out[...] = jnp.dot(h, w3e[...], preferred_element_type=jnp.float32) \
        + b3e[...]


def _scatter_body(nch, rt, rows_hbm, msg_hbm, zeros_hbm, part_out, idr, mbuf,
                  accum, sem):
    cid = lax.axis_index("c")
    sid = lax.axis_index("s")
    wid = sid * NC + cid
    pltpu.sync_copy(zeros_hbm.at[pl.ds(sid * rt, rt)],
                    accum.at[pl.ds(sid * rt, rt)])
    pltpu.sync_copy(rows_hbm.at[pl.ds(wid * nch, nch)], idr)
    plsc.subcore_barrier()

    def chunk(j, carry):
        base = (wid * nch + j) * CHUNK
        pltpu.sync_copy(msg_hbm.at[pl.ds(base, CHUNK)], mbuf)
        pltpu.sync_copy(mbuf, accum.at[idr.at[j]], add=True)
        return carry

    lax.fori_loop(0, nch, chunk, 0)
    plsc.subcore_barrier()
    pltpu.sync_copy(accum.at[pl.ds(sid * rt, rt)],
                    part_out.at[cid, pl.ds(sid * rt, rt)])


def _combine_body(part, out):
    pv = part[...]
    out[...] = pv[0] + pv[1]


def kernel(features, positions, edge_index, W1, b1, W2, b2, W3, b3):
    n, c_in = features.shape
    e = edge_index.shape[1]
    n_sh = 9
    h_dim = W1.shape[1]
    c_out = W3.shape[1] // n_sh
    f32 = jnp.float32

    ei = edge_index.astype(jnp.int32)
    row = ei[0]
    col = ei[1]
    epad = -(-e // (NW * CHUNK)) * (NW * CHUNK)
    nch = epad // (NW * CHUNK)
    pad = epad - e
    rowg = jnp.concatenate([row, jnp.zeros((pad,), jnp.int32)])
    rows = jnp.concatenate([row, jnp.full((pad,), n, jnp.int32)])
    colg = jnp.concatenate([col, jnp.zeros((pad,), jnp.int32)])
    rowg = rowg.reshape(-1, CHUNK)
    rows = rows.reshape(-1, CHUNK)
    colg = colg.reshape(-1, CHUNK)

    npad = -(-(n + 1) // (NS * 8)) * (NS * 8)
    rt = npad // NS
    n8 = -(-n // 8) * 8
    posp = jnp.concatenate(
        [positions.astype(f32), jnp.zeros((n8 - n, 3), f32)], axis=0)
    px = posp[:, 0]
    py = posp[:, 1]
    pz = posp[:, 2]

    w1n = W1[n_sh:]
    w1s = jnp.concatenate([W1[:n_sh], jnp.zeros((16 - n_sh, h_dim), f32)])
    b1r = b1.reshape(1, h_dim)
    b2r = b2.reshape(1, h_dim)
    b3r = b3.reshape(1, n_sh * c_out)

    nb = n // BN
    u_tab, w3e, b3e = pl.pallas_call(
        functools.partial(_prep_body, n_sh, c_out),
        grid=(nb,),
        in_specs=[
            pl.BlockSpec((BN, c_in), lambda i: (i, 0)),
            pl.BlockSpec((c_in, h_dim), lambda i: (0, 0)),
            pl.BlockSpec((1, h_dim), lambda i: (0, 0)),
            pl.BlockSpec((h_dim, n_sh * c_out), lambda i: (0, 0)),
            pl.BlockSpec((1, n_sh * c_out), lambda i: (0, 0)),
        ],
        out_specs=[
            pl.BlockSpec((BN, 128), lambda i: (i, 0)),
            pl.BlockSpec((h_dim, c_out), lambda i: (0, 0)),
            pl.BlockSpec((1, c_out), lambda i: (0, 0)),
        ],
        out_shape=[
            jax.ShapeDtypeStruct((n, 128), f32),
            jax.ShapeDtypeStruct((h_dim, c_out), f32),
            jax.ShapeDtypeStruct((1, c_out), f32),
        ],
    )(features, w1n, b1r, W3, b3r)

    mesh = plsc.VectorSubcoreMesh(core_axis_name="c", subcore_axis_name="s")
    r2t, ss = pl.kernel(
        functools.partial(_gather_body, nch, h_dim),
        out_type=[
            jax.ShapeDtypeStruct((epad, 128), f32),
            jax.ShapeDtypeStruct((8, epad), f32),
        ],
        mesh=mesh,
        scratch_types=[
            pltpu.VMEM((CHUNK,), jnp.int32),
            pltpu.VMEM((CHUNK,), jnp.int32),
            pltpu.VMEM((3, CHUNK), f32),
            pltpu.VMEM((3, CHUNK), f32),
            pltpu.VMEM((CHUNK, 128), f32),
            pltpu.VMEM((8, CHUNK), f32),
            pltpu.SemaphoreType.DMA,
            pltpu.SemaphoreType.DMA,
        ],
    )(rowg, colg, u_tab, px, py, pz)

    msg = pl.pallas_call(
        functools.partial(_mlp_body, h_dim),
        grid=(epad // BE,),
        in_specs=[
            pl.BlockSpec((BE, 128), lambda i: (i, 0)),
            pl.BlockSpec((8, BE), lambda i: (0, i)),
            pl.BlockSpec((16, h_dim), lambda i: (0, 0)),
            pl.BlockSpec((h_dim, h_dim), lambda i: (0, 0)),
            pl.BlockSpec((1, h_dim), lambda i: (0, 0)),
            pl.BlockSpec((h_dim, c_out), lambda i: (0, 0)),
            pl.BlockSpec((1, c_out), lambda i: (0, 0)),
        ],
        out_specs=pl.BlockSpec((BE, c_out), lambda i: (i, 0)),
        out_shape=jax.ShapeDtypeStruct((epad, c_out), f32),
    )(r2t, ss, w1s, W2, b2r, w3e, b3e)

    zeros_init = jnp.zeros((npad, c_out), f32)
    part = pl.kernel(
        functools.partial(_scatter_body, nch, rt),
        out_type=jax.ShapeDtypeStruct((NC, npad, c_out), f32),
        mesh=mesh,
        scratch_types=[
            pltpu.VMEM((nch, CHUNK), jnp.int32),
            pltpu.VMEM((CHUNK, c_out), f32),
            pltpu.VMEM_SHARED((npad, c_out), f32),
            pltpu.SemaphoreType.DMA,
        ],
    )(rows, msg, zeros_init)

    out = pl.pallas_call(
        _combine_body,
        grid=(nb,),
        in_specs=[pl.BlockSpec((NC, BN, c_out), lambda i: (0, i, 0))],
        out_specs=pl.BlockSpec((BN, c_out), lambda i: (i, 0)),
        out_shape=jax.ShapeDtypeStruct((n, c_out), f32),
    )(part)
    return out


# speedup vs baseline: 5.1394x; 1.1166x over previous
import functools
import math

import jax
import jax.numpy as jnp
from jax import lax
from jax.experimental import pallas as pl
from jax.experimental.pallas import tpu as pltpu
from jax.experimental.pallas import tpu_sc as plsc

NC, NS = 2, 16
NW = NC * NS
LANES = 16
CHUNK = 128
BN = 1000
BE = 2048


def _prep_body(n_sh, c_out, feat, w1n, b1, w3, b3, u_out, w3e_out, b3e_out):
    f1 = jnp.dot(feat[...], w1n[...], preferred_element_type=jnp.float32)
    f1 = f1 + b1[...]
    rows = feat.shape[0]
    h_dim = f1.shape[1]
    u_out[...] = jnp.concatenate(
        [f1, jnp.zeros((rows, 128 - h_dim), jnp.float32)], axis=1)
    w3v = w3[...]
    w3e = w3v[:, 0:c_out]
    for k in range(1, n_sh):
        w3e = w3e + w3v[:, k * c_out:(k + 1) * c_out]
    w3e_out[...] = w3e
    b3v = b3[...]
    b3e = b3v[:, 0:c_out]
    for k in range(1, n_sh):
        b3e = b3e + b3v[:, k * c_out:(k + 1) * c_out]
    b3e_out[...] = b3e


def _gather_body(nch, h_dim, rowg, colg, u_hbm, px_hbm, py_hbm, pz_hbm,
                 r2_out, ss_out, idr, idc, pr, pc, bufu, bufs, semg0, semg1):
    cid = lax.axis_index("c")
    sid = lax.axis_index("s")
    wid = sid * NC + cid
    pltpu.sync_copy(rowg.at[pl.ds(wid * nch, nch)], idr)
    pltpu.sync_copy(colg.at[pl.ds(wid * nch, nch)], idc)
    sems = [semg0, semg1]

    def issue(j, b):
        sem = sems[b]
        pltpu.async_copy(u_hbm.at[idc.at[j]], bufu.at[b], sem)
        pltpu.async_copy(px_hbm.at[idr.at[j]], pr.at[b, 0], sem)
        pltpu.async_copy(py_hbm.at[idr.at[j]], pr.at[b, 1], sem)
        pltpu.async_copy(pz_hbm.at[idr.at[j]], pr.at[b, 2], sem)
        pltpu.async_copy(px_hbm.at[idc.at[j]], pc.at[b, 0], sem)
        pltpu.async_copy(py_hbm.at[idc.at[j]], pc.at[b, 1], sem)
        pltpu.async_copy(pz_hbm.at[idc.at[j]], pc.at[b, 2], sem)

    def drain(j, b):
        sem = sems[b]
        base = (wid * nch + j) * CHUNK
        pltpu.make_async_copy(u_hbm.at[idc.at[j]], bufu.at[b], sem).wait()
        pltpu.make_async_copy(px_hbm.at[idr.at[j]], pr.at[b, 0], sem).wait()
        pltpu.make_async_copy(py_hbm.at[idr.at[j]], pr.at[b, 1], sem).wait()
        pltpu.make_async_copy(pz_hbm.at[idr.at[j]], pr.at[b, 2], sem).wait()
        pltpu.make_async_copy(px_hbm.at[idc.at[j]], pc.at[b, 0], sem).wait()
        pltpu.make_async_copy(py_hbm.at[idc.at[j]], pc.at[b, 1], sem).wait()
        pltpu.make_async_copy(pz_hbm.at[idc.at[j]], pc.at[b, 2], sem).wait()
        for k in range(CHUNK // LANES):
            sl = pl.ds(k * LANES, LANES)
            bufs[0, sl] = pr[b, 0, sl] - pc[b, 0, sl]
            bufs[1, sl] = pr[b, 1, sl] - pc[b, 1, sl]
            bufs[2, sl] = pr[b, 2, sl] - pc[b, 2, sl]
        pltpu.sync_copy(bufu.at[b], r2_out.at[pl.ds(base, CHUNK)])
        pltpu.sync_copy(bufs, ss_out.at[:, pl.ds(base, CHUNK)])

    issue(0, 0)

    def body(jj, carry):
        j = 2 * jj
        issue(j + 1, 1)
        drain(j, 0)
        issue(j + 2, 0)
        drain(j + 1, 1)
        return carry

    lax.fori_loop(0, nch // 2 - 1, body, 0)
    issue(nch - 1, 1)
    drain(nch - 2, 0)
    drain(nch - 1, 1)


def _mlp_body(h_dim, r2, ss, w1s, w2, b2, w3e, b3e, out):
    f1 = r2[...][:, 0:h_dim]
    ssv = ss[...]
    dx = ssv[0:1, :]
    dy = ssv[1:2, :]
    dz = ssv[2:3, :]
    n2 = dx * dx + dy * dy + dz * dz
    inv = lax.rsqrt(jnp.maximum(n2, 1e-16))
    x = dx * inv
    y = dy * inv
    z = dz * inv
    p1 = x * x - y * y
    p2 = z * x
    p3 = 2.0 * z * z - x * x - y * y
    ones = jnp.ones_like(x)
    st = jnp.concatenate([x, z, p1, p2, p3, ones], axis=0)
    w1sv = w1s[...]
    c0 = 0.5 / math.sqrt(math.pi)
    cx = 0.5 * math.sqrt(3.0 / (2.0 * math.pi))
    cz = 0.5 * math.sqrt(3.0 / math.pi)
    cxy = 0.25 * math.sqrt(15.0 / (2.0 * math.pi))
    czx = 0.5 * math.sqrt(15.0 / (2.0 * math.pi))
    czz = 0.25 * math.sqrt(5.0 / math.pi)
    us = jnp.concatenate([
        cx * (w1sv[3:4, :] - w1sv[1:2, :]),
        cz * w1sv[2:3, :],
        cxy * (w1sv[4:5, :] + w1sv[8:9, :]),
        czx * (w1sv[7:8, :] - w1sv[5:6, :]),
        czz * w1sv[6:7, :],
        c0 * w1sv[0:1, :],
    ], axis=0)
    hsh = lax.dot_general(st, us, (((0,), (0,)), ((), ())),
                          preferred_element_type=jnp.float32)
    h = f1 + hsh
    h = h * jax.nn.sigmoid(h)
    h = jnp.dot(h, w2[...], preferred_element_type=jnp.float32) + b2[...]
    h = h * jax.nn.sigmoid(h)
    out[...] = jnp.dot(h, w3e[...], preferred_element_type=jnp.float32) \
        + b3e[...]


def _scatter_body(nch, rt, rows_hbm, msg_hbm, zeros_hbm, part_out, idr, mbuf,
                  accum, sem):
    cid = lax.axis_index("c")
    sid = lax.axis_index("s")
    wid = sid * NC + cid
    pltpu.sync_copy(zeros_hbm.at[pl.ds(sid * rt, rt)],
                    accum.at[pl.ds(sid * rt, rt)])
    pltpu.sync_copy(rows_hbm.at[pl.ds(wid * nch, nch)], idr)
    plsc.subcore_barrier()

    def chunk(j, carry):
        base = (wid * nch + j) * CHUNK
        pltpu.sync_copy(msg_hbm.at[pl.ds(base, CHUNK)], mbuf)
        pltpu.sync_copy(mbuf, accum.at[idr.at[j]], add=True)
        return carry

    lax.fori_loop(0, nch, chunk, 0)
    plsc.subcore_barrier()
    pltpu.sync_copy(accum.at[pl.ds(sid * rt, rt)],
                    part_out.at[cid, pl.ds(sid * rt, rt)])


def _combine_body(part, out):
    pv = part[...]
    out[...] = pv[0] + pv[1]


def kernel(features, positions, edge_index, W1, b1, W2, b2, W3, b3):
    n, c_in = features.shape
    e = edge_index.shape[1]
    n_sh = 9
    h_dim = W1.shape[1]
    c_out = W3.shape[1] // n_sh
    f32 = jnp.float32

    ei = edge_index.astype(jnp.int32)
    row = ei[0]
    col = ei[1]
    epad = -(-e // (NW * CHUNK)) * (NW * CHUNK)
    nch = epad // (NW * CHUNK)
    pad = epad - e
    rowg = jnp.concatenate([row, jnp.zeros((pad,), jnp.int32)])
    rows = jnp.concatenate([row, jnp.full((pad,), n, jnp.int32)])
    colg = jnp.concatenate([col, jnp.zeros((pad,), jnp.int32)])
    rowg = rowg.reshape(-1, CHUNK)
    rows = rows.reshape(-1, CHUNK)
    colg = colg.reshape(-1, CHUNK)

    npad = -(-(n + 1) // (NS * 8)) * (NS * 8)
    rt = npad // NS
    n8 = -(-n // 8) * 8
    posp = jnp.concatenate(
        [positions.astype(f32), jnp.zeros((n8 - n, 3), f32)], axis=0)
    px = posp[:, 0]
    py = posp[:, 1]
    pz = posp[:, 2]

    w1n = W1[n_sh:]
    w1s = jnp.concatenate([W1[:n_sh], jnp.zeros((16 - n_sh, h_dim), f32)])
    b1r = b1.reshape(1, h_dim)
    b2r = b2.reshape(1, h_dim)
    b3r = b3.reshape(1, n_sh * c_out)

    nb = n // BN
    u_tab, w3e, b3e = pl.pallas_call(
        functools.partial(_prep_body, n_sh, c_out),
        grid=(nb,),
        in_specs=[
            pl.BlockSpec((BN, c_in), lambda i: (i, 0)),
            pl.BlockSpec((c_in, h_dim), lambda i: (0, 0)),
            pl.BlockSpec((1, h_dim), lambda i: (0, 0)),
            pl.BlockSpec((h_dim, n_sh * c_out), lambda i: (0, 0)),
            pl.BlockSpec((1, n_sh * c_out), lambda i: (0, 0)),
        ],
        out_specs=[
            pl.BlockSpec((BN, 128), lambda i: (i, 0)),
            pl.BlockSpec((h_dim, c_out), lambda i: (0, 0)),
            pl.BlockSpec((1, c_out), lambda i: (0, 0)),
        ],
        out_shape=[
            jax.ShapeDtypeStruct((n, 128), f32),
            jax.ShapeDtypeStruct((h_dim, c_out), f32),
            jax.ShapeDtypeStruct((1, c_out), f32),
        ],
    )(features, w1n, b1r, W3, b3r)

    mesh = plsc.VectorSubcoreMesh(core_axis_name="c", subcore_axis_name="s")
    r2t, ss = pl.kernel(
        functools.partial(_gather_body, nch, h_dim),
        out_type=[
            jax.ShapeDtypeStruct((epad, 128), f32),
            jax.ShapeDtypeStruct((8, epad), f32),
        ],
        mesh=mesh,
        scratch_types=[
            pltpu.VMEM((nch, CHUNK), jnp.int32),
            pltpu.VMEM((nch, CHUNK), jnp.int32),
            pltpu.VMEM((2, 3, CHUNK), f32),
            pltpu.VMEM((2, 3, CHUNK), f32),
            pltpu.VMEM((2, CHUNK, 128), f32),
            pltpu.VMEM((8, CHUNK), f32),
            pltpu.SemaphoreType.DMA,
            pltpu.SemaphoreType.DMA,
        ],
    )(rowg, colg, u_tab, px, py, pz)

    msg = pl.pallas_call(
        functools.partial(_mlp_body, h_dim),
        grid=(epad // BE,),
        in_specs=[
            pl.BlockSpec((BE, 128), lambda i: (i, 0)),
            pl.BlockSpec((8, BE), lambda i: (0, i)),
            pl.BlockSpec((16, h_dim), lambda i: (0, 0)),
            pl.BlockSpec((h_dim, h_dim), lambda i: (0, 0)),
            pl.BlockSpec((1, h_dim), lambda i: (0, 0)),
            pl.BlockSpec((h_dim, c_out), lambda i: (0, 0)),
            pl.BlockSpec((1, c_out), lambda i: (0, 0)),
        ],
        out_specs=pl.BlockSpec((BE, c_out), lambda i: (i, 0)),
        out_shape=jax.ShapeDtypeStruct((epad, c_out), f32),
    )(r2t, ss, w1s, W2, b2r, w3e, b3e)

    zeros_init = jnp.zeros((npad, c_out), f32)
    part = pl.kernel(
        functools.partial(_scatter_body, nch, rt),
        out_type=jax.ShapeDtypeStruct((NC, npad, c_out), f32),
        mesh=mesh,
        scratch_types=[
            pltpu.VMEM((nch, CHUNK), jnp.int32),
            pltpu.VMEM((CHUNK, c_out), f32),
            pltpu.VMEM_SHARED((npad, c_out), f32),
            pltpu.SemaphoreType.DMA,
        ],
    )(rows, msg, zeros_init)

    out = pl.pallas_call(
        _combine_body,
        grid=(nb,),
        in_specs=[pl.BlockSpec((NC, BN, c_out), lambda i: (0, i, 0))],
        out_specs=pl.BlockSpec((BN, c_out), lambda i: (i, 0)),
        out_shape=jax.ShapeDtypeStruct((n, c_out), f32),
    )(part)
    return out


# speedup vs baseline: 8.6502x; 1.6831x over previous
import functools
import math

import jax
import jax.numpy as jnp
from jax import lax
from jax.experimental import pallas as pl
from jax.experimental.pallas import tpu as pltpu
from jax.experimental.pallas import tpu_sc as plsc

NC, NS = 2, 16
NW = NC * NS
LANES = 16
CHUNK = 128
BN = 1000
BE = 2048


def _prep_body(n_sh, c_out, feat, w1n, b1, w3, b3, u_out, w3e_out, b3e_out):
    f1 = jnp.dot(feat[...], w1n[...], preferred_element_type=jnp.float32)
    f1 = f1 + b1[...]
    rows = feat.shape[0]
    h_dim = f1.shape[1]
    u_out[...] = jnp.concatenate(
        [f1, jnp.zeros((rows, 128 - h_dim), jnp.float32)], axis=1)
    w3v = w3[...]
    w3e = w3v[:, 0:c_out]
    for k in range(1, n_sh):
        w3e = w3e + w3v[:, k * c_out:(k + 1) * c_out]
    w3e_out[...] = w3e
    b3v = b3[...]
    b3e = b3v[:, 0:c_out]
    for k in range(1, n_sh):
        b3e = b3e + b3v[:, k * c_out:(k + 1) * c_out]
    b3e_out[...] = b3e


def _gather_body(nch, h_dim, rowg, colg, u_hbm, px_hbm, py_hbm, pz_hbm,
                 r2_out, ss_out, idr, idc, pr, pc, bufu, bufs, u_sp, pxs,
                 pys, pzs, semg0, semg1):
    cid = lax.axis_index("c")
    sid = lax.axis_index("s")
    wid = sid * NC + cid

    @pl.when(sid == 0)
    def _stage():
        pltpu.sync_copy(u_hbm, u_sp)
        pltpu.sync_copy(px_hbm, pxs)
        pltpu.sync_copy(py_hbm, pys)
        pltpu.sync_copy(pz_hbm, pzs)

    pltpu.sync_copy(rowg.at[pl.ds(wid * nch, nch)], idr)
    pltpu.sync_copy(colg.at[pl.ds(wid * nch, nch)], idc)
    plsc.subcore_barrier()
    sems = [semg0, semg1]

    def issue(j, b):
        sem = sems[b]
        pltpu.async_copy(u_sp.at[idc.at[j]], bufu.at[b], sem)
        pltpu.async_copy(pxs.at[idr.at[j]], pr.at[b, 0], sem)
        pltpu.async_copy(pys.at[idr.at[j]], pr.at[b, 1], sem)
        pltpu.async_copy(pzs.at[idr.at[j]], pr.at[b, 2], sem)
        pltpu.async_copy(pxs.at[idc.at[j]], pc.at[b, 0], sem)
        pltpu.async_copy(pys.at[idc.at[j]], pc.at[b, 1], sem)
        pltpu.async_copy(pzs.at[idc.at[j]], pc.at[b, 2], sem)

    def drain(j, b):
        sem = sems[b]
        base = (wid * nch + j) * CHUNK
        pltpu.make_async_copy(u_sp.at[idc.at[j]], bufu.at[b], sem).wait()
        pltpu.make_async_copy(pxs.at[idr.at[j]], pr.at[b, 0], sem).wait()
        pltpu.make_async_copy(pys.at[idr.at[j]], pr.at[b, 1], sem).wait()
        pltpu.make_async_copy(pzs.at[idr.at[j]], pr.at[b, 2], sem).wait()
        pltpu.make_async_copy(pxs.at[idc.at[j]], pc.at[b, 0], sem).wait()
        pltpu.make_async_copy(pys.at[idc.at[j]], pc.at[b, 1], sem).wait()
        pltpu.make_async_copy(pzs.at[idc.at[j]], pc.at[b, 2], sem).wait()
        for k in range(CHUNK // LANES):
            sl = pl.ds(k * LANES, LANES)
            bufs[0, sl] = pr[b, 0, sl] - pc[b, 0, sl]
            bufs[1, sl] = pr[b, 1, sl] - pc[b, 1, sl]
            bufs[2, sl] = pr[b, 2, sl] - pc[b, 2, sl]
        pltpu.sync_copy(bufu.at[b], r2_out.at[pl.ds(base, CHUNK)])
        pltpu.sync_copy(bufs, ss_out.at[:, pl.ds(base, CHUNK)])

    issue(0, 0)

    def body(jj, carry):
        j = 2 * jj
        issue(j + 1, 1)
        drain(j, 0)
        issue(j + 2, 0)
        drain(j + 1, 1)
        return carry

    lax.fori_loop(0, nch // 2 - 1, body, 0)
    issue(nch - 1, 1)
    drain(nch - 2, 0)
    drain(nch - 1, 1)


def _mlp_body(h_dim, r2, ss, w1s, w2, b2, w3e, b3e, out):
    f1 = r2[...][:, 0:h_dim]
    ssv = ss[...]
    dx = ssv[0:1, :]
    dy = ssv[1:2, :]
    dz = ssv[2:3, :]
    n2 = dx * dx + dy * dy + dz * dz
    inv = lax.rsqrt(jnp.maximum(n2, 1e-16))
    x = dx * inv
    y = dy * inv
    z = dz * inv
    p1 = x * x - y * y
    p2 = z * x
    p3 = 2.0 * z * z - x * x - y * y
    ones = jnp.ones_like(x)
    st = jnp.concatenate([x, z, p1, p2, p3, ones], axis=0)
    w1sv = w1s[...]
    c0 = 0.5 / math.sqrt(math.pi)
    cx = 0.5 * math.sqrt(3.0 / (2.0 * math.pi))
    cz = 0.5 * math.sqrt(3.0 / math.pi)
    cxy = 0.25 * math.sqrt(15.0 / (2.0 * math.pi))
    czx = 0.5 * math.sqrt(15.0 / (2.0 * math.pi))
    czz = 0.25 * math.sqrt(5.0 / math.pi)
    us = jnp.concatenate([
        cx * (w1sv[3:4, :] - w1sv[1:2, :]),
        cz * w1sv[2:3, :],
        cxy * (w1sv[4:5, :] + w1sv[8:9, :]),
        czx * (w1sv[7:8, :] - w1sv[5:6, :]),
        czz * w1sv[6:7, :],
        c0 * w1sv[0:1, :],
    ], axis=0)
    hsh = lax.dot_general(st, us, (((0,), (0,)), ((), ())),
                          preferred_element_type=jnp.float32)
    h = f1 + hsh
    h = h * jax.nn.sigmoid(h)
    h = jnp.dot(h, w2[...], preferred_element_type=jnp.float32) + b2[...]
    h = h * jax.nn.sigmoid(h)
    out[...] = jnp.dot(h, w3e[...], preferred_element_type=jnp.float32) \
        + b3e[...]


def _scatter_body(nch, rt, rows_hbm, msg_hbm, zeros_hbm, part_out, idr, mbuf,
                  accum, sem):
    cid = lax.axis_index("c")
    sid = lax.axis_index("s")
    wid = sid * NC + cid
    pltpu.sync_copy(zeros_hbm.at[pl.ds(sid * rt, rt)],
                    accum.at[pl.ds(sid * rt, rt)])
    pltpu.sync_copy(rows_hbm.at[pl.ds(wid * nch, nch)], idr)
    plsc.subcore_barrier()

    def chunk(j, carry):
        base = (wid * nch + j) * CHUNK
        pltpu.sync_copy(msg_hbm.at[pl.ds(base, CHUNK)], mbuf)
        pltpu.sync_copy(mbuf, accum.at[idr.at[j]], add=True)
        return carry

    lax.fori_loop(0, nch, chunk, 0)
    plsc.subcore_barrier()
    pltpu.sync_copy(accum.at[pl.ds(sid * rt, rt)],
                    part_out.at[cid, pl.ds(sid * rt, rt)])


def _combine_body(part, out):
    pv = part[...]
    out[...] = pv[0] + pv[1]


def kernel(features, positions, edge_index, W1, b1, W2, b2, W3, b3):
    n, c_in = features.shape
    e = edge_index.shape[1]
    n_sh = 9
    h_dim = W1.shape[1]
    c_out = W3.shape[1] // n_sh
    f32 = jnp.float32

    ei = edge_index.astype(jnp.int32)
    row = ei[0]
    col = ei[1]
    epad = -(-e // (NW * CHUNK)) * (NW * CHUNK)
    nch = epad // (NW * CHUNK)
    pad = epad - e
    rowg = jnp.concatenate([row, jnp.zeros((pad,), jnp.int32)])
    rows = jnp.concatenate([row, jnp.full((pad,), n, jnp.int32)])
    colg = jnp.concatenate([col, jnp.zeros((pad,), jnp.int32)])
    rowg = rowg.reshape(-1, CHUNK)
    rows = rows.reshape(-1, CHUNK)
    colg = colg.reshape(-1, CHUNK)

    npad = -(-(n + 1) // (NS * 8)) * (NS * 8)
    rt = npad // NS
    n8 = -(-n // 8) * 8
    posp = jnp.concatenate(
        [positions.astype(f32), jnp.zeros((n8 - n, 3), f32)], axis=0)
    px = posp[:, 0]
    py = posp[:, 1]
    pz = posp[:, 2]

    w1n = W1[n_sh:]
    w1s = jnp.concatenate([W1[:n_sh], jnp.zeros((16 - n_sh, h_dim), f32)])
    b1r = b1.reshape(1, h_dim)
    b2r = b2.reshape(1, h_dim)
    b3r = b3.reshape(1, n_sh * c_out)

    nb = n // BN
    u_tab, w3e, b3e = pl.pallas_call(
        functools.partial(_prep_body, n_sh, c_out),
        grid=(nb,),
        in_specs=[
            pl.BlockSpec((BN, c_in), lambda i: (i, 0)),
            pl.BlockSpec((c_in, h_dim), lambda i: (0, 0)),
            pl.BlockSpec((1, h_dim), lambda i: (0, 0)),
            pl.BlockSpec((h_dim, n_sh * c_out), lambda i: (0, 0)),
            pl.BlockSpec((1, n_sh * c_out), lambda i: (0, 0)),
        ],
        out_specs=[
            pl.BlockSpec((BN, 128), lambda i: (i, 0)),
            pl.BlockSpec((h_dim, c_out), lambda i: (0, 0)),
            pl.BlockSpec((1, c_out), lambda i: (0, 0)),
        ],
        out_shape=[
            jax.ShapeDtypeStruct((n, 128), f32),
            jax.ShapeDtypeStruct((h_dim, c_out), f32),
            jax.ShapeDtypeStruct((1, c_out), f32),
        ],
    )(features, w1n, b1r, W3, b3r)

    mesh = plsc.VectorSubcoreMesh(core_axis_name="c", subcore_axis_name="s")
    r2t, ss = pl.kernel(
        functools.partial(_gather_body, nch, h_dim),
        out_type=[
            jax.ShapeDtypeStruct((epad, 128), f32),
            jax.ShapeDtypeStruct((8, epad), f32),
        ],
        mesh=mesh,
        scratch_types=[
            pltpu.VMEM((nch, CHUNK), jnp.int32),
            pltpu.VMEM((nch, CHUNK), jnp.int32),
            pltpu.VMEM((2, 3, CHUNK), f32),
            pltpu.VMEM((2, 3, CHUNK), f32),
            pltpu.VMEM((2, CHUNK, 128), f32),
            pltpu.VMEM((8, CHUNK), f32),
            pltpu.VMEM_SHARED((n, 128), f32),
            pltpu.VMEM_SHARED((n8,), f32),
            pltpu.VMEM_SHARED((n8,), f32),
            pltpu.VMEM_SHARED((n8,), f32),
            pltpu.SemaphoreType.DMA,
            pltpu.SemaphoreType.DMA,
        ],
    )(rowg, colg, u_tab, px, py, pz)

    msg = pl.pallas_call(
        functools.partial(_mlp_body, h_dim),
        grid=(epad // BE,),
        in_specs=[
            pl.BlockSpec((BE, 128), lambda i: (i, 0)),
            pl.BlockSpec((8, BE), lambda i: (0, i)),
            pl.BlockSpec((16, h_dim), lambda i: (0, 0)),
            pl.BlockSpec((h_dim, h_dim), lambda i: (0, 0)),
            pl.BlockSpec((1, h_dim), lambda i: (0, 0)),
            pl.BlockSpec((h_dim, c_out), lambda i: (0, 0)),
            pl.BlockSpec((1, c_out), lambda i: (0, 0)),
        ],
        out_specs=pl.BlockSpec((BE, c_out), lambda i: (i, 0)),
        out_shape=jax.ShapeDtypeStruct((epad, c_out), f32),
    )(r2t, ss, w1s, W2, b2r, w3e, b3e)

    zeros_init = jnp.zeros((npad, c_out), f32)
    part = pl.kernel(
        functools.partial(_scatter_body, nch, rt),
        out_type=jax.ShapeDtypeStruct((NC, npad, c_out), f32),
        mesh=mesh,
        scratch_types=[
            pltpu.VMEM((nch, CHUNK), jnp.int32),
            pltpu.VMEM((CHUNK, c_out), f32),
            pltpu.VMEM_SHARED((npad, c_out), f32),
            pltpu.SemaphoreType.DMA,
        ],
    )(rows, msg, zeros_init)

    out = pl.pallas_call(
        _combine_body,
        grid=(nb,),
        in_specs=[pl.BlockSpec((NC, BN, c_out), lambda i: (0, i, 0))],
        out_specs=pl.BlockSpec((BN, c_out), lambda i: (i, 0)),
        out_shape=jax.ShapeDtypeStruct((n, c_out), f32),
    )(part)
    return out


# speedup vs baseline: 10.6493x; 1.2311x over previous
import functools
import math

import jax
import jax.numpy as jnp
from jax import lax
from jax.experimental import pallas as pl
from jax.experimental.pallas import tpu as pltpu
from jax.experimental.pallas import tpu_sc as plsc

NC, NS = 2, 16
NW = NC * NS
LANES = 16
CHUNK = 128
BN = 1000
BE = 4096


def _prep_body(n_sh, c_out, feat, w1n, b1, w3, b3, u_out, w3e_out, b3e_out):
    f1 = jnp.dot(feat[...], w1n[...], preferred_element_type=jnp.float32)
    f1 = f1 + b1[...]
    rows = feat.shape[0]
    h_dim = f1.shape[1]
    u_out[...] = jnp.concatenate(
        [f1, jnp.zeros((rows, 128 - h_dim), jnp.float32)], axis=1)
    w3v = w3[...]
    w3e = w3v[:, 0:c_out]
    for k in range(1, n_sh):
        w3e = w3e + w3v[:, k * c_out:(k + 1) * c_out]
    w3e_out[...] = w3e
    b3v = b3[...]
    b3e = b3v[:, 0:c_out]
    for k in range(1, n_sh):
        b3e = b3e + b3v[:, k * c_out:(k + 1) * c_out]
    b3e_out[...] = b3e


def _gather_body(nch, h_dim, rowg, colg, u_hbm, px_hbm, py_hbm, pz_hbm,
                 r2_out, ss_out, idr, idc, pr, pc, bufu, bufs, u_sp, pxs,
                 pys, pzs, semg0, semg1):
    cid = lax.axis_index("c")
    sid = lax.axis_index("s")
    wid = sid * NC + cid

    @pl.when(sid == 0)
    def _stage():
        pltpu.sync_copy(u_hbm, u_sp)
        pltpu.sync_copy(px_hbm, pxs)
        pltpu.sync_copy(py_hbm, pys)
        pltpu.sync_copy(pz_hbm, pzs)

    pltpu.sync_copy(rowg.at[pl.ds(wid * nch, nch)], idr)
    pltpu.sync_copy(colg.at[pl.ds(wid * nch, nch)], idc)
    plsc.subcore_barrier()
    sems = [semg0, semg1]

    def issue(j, b):
        sem = sems[b]
        pltpu.async_copy(u_sp.at[idc.at[j]], bufu.at[b], sem)
        pltpu.async_copy(pxs.at[idr.at[j]], pr.at[b, 0], sem)
        pltpu.async_copy(pys.at[idr.at[j]], pr.at[b, 1], sem)
        pltpu.async_copy(pzs.at[idr.at[j]], pr.at[b, 2], sem)
        pltpu.async_copy(pxs.at[idc.at[j]], pc.at[b, 0], sem)
        pltpu.async_copy(pys.at[idc.at[j]], pc.at[b, 1], sem)
        pltpu.async_copy(pzs.at[idc.at[j]], pc.at[b, 2], sem)

    def drain(j, b):
        sem = sems[b]
        base = (wid * nch + j) * CHUNK
        pltpu.make_async_copy(u_sp.at[idc.at[j]], bufu.at[b], sem).wait()
        pltpu.make_async_copy(pxs.at[idr.at[j]], pr.at[b, 0], sem).wait()
        pltpu.make_async_copy(pys.at[idr.at[j]], pr.at[b, 1], sem).wait()
        pltpu.make_async_copy(pzs.at[idr.at[j]], pr.at[b, 2], sem).wait()
        pltpu.make_async_copy(pxs.at[idc.at[j]], pc.at[b, 0], sem).wait()
        pltpu.make_async_copy(pys.at[idc.at[j]], pc.at[b, 1], sem).wait()
        pltpu.make_async_copy(pzs.at[idc.at[j]], pc.at[b, 2], sem).wait()
        for k in range(CHUNK // LANES):
            sl = pl.ds(k * LANES, LANES)
            bufs[0, sl] = pr[b, 0, sl] - pc[b, 0, sl]
            bufs[1, sl] = pr[b, 1, sl] - pc[b, 1, sl]
            bufs[2, sl] = pr[b, 2, sl] - pc[b, 2, sl]
        pltpu.sync_copy(bufu.at[b], r2_out.at[pl.ds(base, CHUNK)])
        pltpu.sync_copy(bufs, ss_out.at[:, pl.ds(base, CHUNK)])

    issue(0, 0)

    def body(jj, carry):
        j = 2 * jj
        issue(j + 1, 1)
        drain(j, 0)
        issue(j + 2, 0)
        drain(j + 1, 1)
        return carry

    lax.fori_loop(0, nch // 2 - 1, body, 0)
    issue(nch - 1, 1)
    drain(nch - 2, 0)
    drain(nch - 1, 1)


def _mlp_body(h_dim, r2, ss, w1s, w2, b2, w3e, b3e, out):
    f1 = r2[...][:, 0:h_dim]
    ssv = ss[...]
    dx = ssv[0:1, :]
    dy = ssv[1:2, :]
    dz = ssv[2:3, :]
    n2 = dx * dx + dy * dy + dz * dz
    inv = lax.rsqrt(jnp.maximum(n2, 1e-16))
    x = dx * inv
    y = dy * inv
    z = dz * inv
    p1 = x * x - y * y
    p2 = z * x
    p3 = 2.0 * z * z - x * x - y * y
    ones = jnp.ones_like(x)
    st = jnp.concatenate([x, z, p1, p2, p3, ones], axis=0)
    w1sv = w1s[...]
    c0 = 0.5 / math.sqrt(math.pi)
    cx = 0.5 * math.sqrt(3.0 / (2.0 * math.pi))
    cz = 0.5 * math.sqrt(3.0 / math.pi)
    cxy = 0.25 * math.sqrt(15.0 / (2.0 * math.pi))
    czx = 0.5 * math.sqrt(15.0 / (2.0 * math.pi))
    czz = 0.25 * math.sqrt(5.0 / math.pi)
    us = jnp.concatenate([
        cx * (w1sv[3:4, :] - w1sv[1:2, :]),
        cz * w1sv[2:3, :],
        cxy * (w1sv[4:5, :] + w1sv[8:9, :]),
        czx * (w1sv[7:8, :] - w1sv[5:6, :]),
        czz * w1sv[6:7, :],
        c0 * w1sv[0:1, :],
    ], axis=0)
    hsh = lax.dot_general(st, us, (((0,), (0,)), ((), ())),
                          preferred_element_type=jnp.float32)
    h = f1 + hsh
    h = h * jax.nn.sigmoid(h)
    h = jnp.dot(h, w2[...], preferred_element_type=jnp.float32) + b2[...]
    h = h * jax.nn.sigmoid(h)
    out[...] = jnp.dot(h, w3e[...], preferred_element_type=jnp.float32) \
        + b3e[...]


def _scatter_body(nch, rt, rows_hbm, msg_hbm, zeros_hbm, part_out, idr, mbuf,
                  accum, semr0, semr1):
    cid = lax.axis_index("c")
    sid = lax.axis_index("s")
    wid = sid * NC + cid
    pltpu.sync_copy(zeros_hbm.at[pl.ds(sid * rt, rt)],
                    accum.at[pl.ds(sid * rt, rt)])
    pltpu.sync_copy(rows_hbm.at[pl.ds(wid * nch, nch)], idr)
    plsc.subcore_barrier()
    sems = [semr0, semr1]

    def rd(j, b):
        base = (wid * nch + j) * CHUNK
        pltpu.async_copy(msg_hbm.at[pl.ds(base, CHUNK)], mbuf.at[b], sems[b])

    def sc(j, b):
        base = (wid * nch + j) * CHUNK
        pltpu.make_async_copy(msg_hbm.at[pl.ds(base, CHUNK)], mbuf.at[b],
                              sems[b]).wait()
        pltpu.sync_copy(mbuf.at[b], accum.at[idr.at[j]], add=True)

    rd(0, 0)

    def body(jj, carry):
        j = 2 * jj
        rd(j + 1, 1)
        sc(j, 0)
        rd(j + 2, 0)
        sc(j + 1, 1)
        return carry

    lax.fori_loop(0, nch // 2 - 1, body, 0)
    rd(nch - 1, 1)
    sc(nch - 2, 0)
    sc(nch - 1, 1)
    plsc.subcore_barrier()
    pltpu.sync_copy(accum.at[pl.ds(sid * rt, rt)],
                    part_out.at[cid, pl.ds(sid * rt, rt)])


def _combine_body(part, out):
    pv = part[...]
    out[...] = pv[0] + pv[1]


def kernel(features, positions, edge_index, W1, b1, W2, b2, W3, b3):
    n, c_in = features.shape
    e = edge_index.shape[1]
    n_sh = 9
    h_dim = W1.shape[1]
    c_out = W3.shape[1] // n_sh
    f32 = jnp.float32

    ei = edge_index.astype(jnp.int32)
    row = ei[0]
    col = ei[1]
    epad = -(-e // (NW * CHUNK)) * (NW * CHUNK)
    nch = epad // (NW * CHUNK)
    pad = epad - e
    rowg = jnp.concatenate([row, jnp.zeros((pad,), jnp.int32)])
    rows = jnp.concatenate([row, jnp.full((pad,), n, jnp.int32)])
    colg = jnp.concatenate([col, jnp.zeros((pad,), jnp.int32)])
    rowg = rowg.reshape(-1, CHUNK)
    rows = rows.reshape(-1, CHUNK)
    colg = colg.reshape(-1, CHUNK)

    npad = -(-(n + 1) // (NS * 8)) * (NS * 8)
    rt = npad // NS
    n8 = -(-n // 8) * 8
    posp = jnp.concatenate(
        [positions.astype(f32), jnp.zeros((n8 - n, 3), f32)], axis=0)
    px = posp[:, 0]
    py = posp[:, 1]
    pz = posp[:, 2]

    w1n = W1[n_sh:]
    w1s = jnp.concatenate([W1[:n_sh], jnp.zeros((16 - n_sh, h_dim), f32)])
    b1r = b1.reshape(1, h_dim)
    b2r = b2.reshape(1, h_dim)
    b3r = b3.reshape(1, n_sh * c_out)

    nb = n // BN
    u_tab, w3e, b3e = pl.pallas_call(
        functools.partial(_prep_body, n_sh, c_out),
        grid=(nb,),
        in_specs=[
            pl.BlockSpec((BN, c_in), lambda i: (i, 0)),
            pl.BlockSpec((c_in, h_dim), lambda i: (0, 0)),
            pl.BlockSpec((1, h_dim), lambda i: (0, 0)),
            pl.BlockSpec((h_dim, n_sh * c_out), lambda i: (0, 0)),
            pl.BlockSpec((1, n_sh * c_out), lambda i: (0, 0)),
        ],
        out_specs=[
            pl.BlockSpec((BN, 128), lambda i: (i, 0)),
            pl.BlockSpec((h_dim, c_out), lambda i: (0, 0)),
            pl.BlockSpec((1, c_out), lambda i: (0, 0)),
        ],
        out_shape=[
            jax.ShapeDtypeStruct((n, 128), f32),
            jax.ShapeDtypeStruct((h_dim, c_out), f32),
            jax.ShapeDtypeStruct((1, c_out), f32),
        ],
    )(features, w1n, b1r, W3, b3r)

    mesh = plsc.VectorSubcoreMesh(core_axis_name="c", subcore_axis_name="s")
    r2t, ss = pl.kernel(
        functools.partial(_gather_body, nch, h_dim),
        out_type=[
            jax.ShapeDtypeStruct((epad, 128), f32),
            jax.ShapeDtypeStruct((8, epad), f32),
        ],
        mesh=mesh,
        scratch_types=[
            pltpu.VMEM((nch, CHUNK), jnp.int32),
            pltpu.VMEM((nch, CHUNK), jnp.int32),
            pltpu.VMEM((2, 3, CHUNK), f32),
            pltpu.VMEM((2, 3, CHUNK), f32),
            pltpu.VMEM((2, CHUNK, 128), f32),
            pltpu.VMEM((8, CHUNK), f32),
            pltpu.VMEM_SHARED((n, 128), f32),
            pltpu.VMEM_SHARED((n8,), f32),
            pltpu.VMEM_SHARED((n8,), f32),
            pltpu.VMEM_SHARED((n8,), f32),
            pltpu.SemaphoreType.DMA,
            pltpu.SemaphoreType.DMA,
        ],
    )(rowg, colg, u_tab, px, py, pz)

    msg = pl.pallas_call(
        functools.partial(_mlp_body, h_dim),
        grid=(epad // BE,),
        in_specs=[
            pl.BlockSpec((BE, 128), lambda i: (i, 0)),
            pl.BlockSpec((8, BE), lambda i: (0, i)),
            pl.BlockSpec((16, h_dim), lambda i: (0, 0)),
            pl.BlockSpec((h_dim, h_dim), lambda i: (0, 0)),
            pl.BlockSpec((1, h_dim), lambda i: (0, 0)),
            pl.BlockSpec((h_dim, c_out), lambda i: (0, 0)),
            pl.BlockSpec((1, c_out), lambda i: (0, 0)),
        ],
        out_specs=pl.BlockSpec((BE, c_out), lambda i: (i, 0)),
        out_shape=jax.ShapeDtypeStruct((epad, c_out), f32),
    )(r2t, ss, w1s, W2, b2r, w3e, b3e)

    zeros_init = jnp.zeros((npad, c_out), f32)
    part = pl.kernel(
        functools.partial(_scatter_body, nch, rt),
        out_type=jax.ShapeDtypeStruct((NC, npad, c_out), f32),
        mesh=mesh,
        scratch_types=[
            pltpu.VMEM((nch, CHUNK), jnp.int32),
            pltpu.VMEM((2, CHUNK, c_out), f32),
            pltpu.VMEM_SHARED((npad, c_out), f32),
            pltpu.SemaphoreType.DMA,
            pltpu.SemaphoreType.DMA,
        ],
    )(rows, msg, zeros_init)

    out = pl.pallas_call(
        _combine_body,
        grid=(nb,),
        in_specs=[pl.BlockSpec((NC, BN, c_out), lambda i: (0, i, 0))],
        out_specs=pl.BlockSpec((BN, c_out), lambda i: (i, 0)),
        out_shape=jax.ShapeDtypeStruct((n, c_out), f32),
    )(part)
    return out


# speedup vs baseline: 11.4050x; 1.0710x over previous
import functools
import math

import jax
import jax.numpy as jnp
from jax import lax
from jax.experimental import pallas as pl
from jax.experimental.pallas import tpu as pltpu
from jax.experimental.pallas import tpu_sc as plsc

NC, NS = 2, 16
NW = NC * NS
LANES = 16
CHUNK = 128
BN = 1000
BE = 4096


def _prep_body(n_sh, c_out, feat, w1n, b1, w3, b3, u_out, w3e_out, b3e_out):
    f1 = jnp.dot(feat[...], w1n[...], preferred_element_type=jnp.float32)
    f1 = f1 + b1[...]
    rows = feat.shape[0]
    h_dim = f1.shape[1]
    u_out[...] = jnp.concatenate(
        [f1, jnp.zeros((rows, 128 - h_dim), jnp.float32)], axis=1)
    w3v = w3[...]
    w3e = w3v[:, 0:c_out]
    for k in range(1, n_sh):
        w3e = w3e + w3v[:, k * c_out:(k + 1) * c_out]
    w3e_out[...] = w3e
    b3v = b3[...]
    b3e = b3v[:, 0:c_out]
    for k in range(1, n_sh):
        b3e = b3e + b3v[:, k * c_out:(k + 1) * c_out]
    b3e_out[...] = b3e


def _gather_body(nch_tot, off, nch, h_dim, rowg, colg, u_hbm, px_hbm, py_hbm,
                 pz_hbm, r2_out, ss_out, idr, idc, pr, pc, bufu, bufs, u_sp,
                 pxs, pys, pzs, semg0, semg1):
    cid = lax.axis_index("c")
    sid = lax.axis_index("s")
    wid = sid * NC + cid

    @pl.when(sid == 0)
    def _stage():
        pltpu.sync_copy(u_hbm, u_sp)
        pltpu.sync_copy(px_hbm, pxs)
        pltpu.sync_copy(py_hbm, pys)
        pltpu.sync_copy(pz_hbm, pzs)

    pltpu.sync_copy(rowg.at[pl.ds(wid * nch_tot + off, nch)], idr)
    pltpu.sync_copy(colg.at[pl.ds(wid * nch_tot + off, nch)], idc)
    plsc.subcore_barrier()
    sems = [semg0, semg1]

    def issue(j, b):
        sem = sems[b]
        pltpu.async_copy(u_sp.at[idc.at[j]], bufu.at[b], sem)
        pltpu.async_copy(pxs.at[idr.at[j]], pr.at[b, 0], sem)
        pltpu.async_copy(pys.at[idr.at[j]], pr.at[b, 1], sem)
        pltpu.async_copy(pzs.at[idr.at[j]], pr.at[b, 2], sem)
        pltpu.async_copy(pxs.at[idc.at[j]], pc.at[b, 0], sem)
        pltpu.async_copy(pys.at[idc.at[j]], pc.at[b, 1], sem)
        pltpu.async_copy(pzs.at[idc.at[j]], pc.at[b, 2], sem)

    def drain(j, b):
        sem = sems[b]
        base = (wid * nch + j) * CHUNK
        pltpu.make_async_copy(u_sp.at[idc.at[j]], bufu.at[b], sem).wait()
        pltpu.make_async_copy(pxs.at[idr.at[j]], pr.at[b, 0], sem).wait()
        pltpu.make_async_copy(pys.at[idr.at[j]], pr.at[b, 1], sem).wait()
        pltpu.make_async_copy(pzs.at[idr.at[j]], pr.at[b, 2], sem).wait()
        pltpu.make_async_copy(pxs.at[idc.at[j]], pc.at[b, 0], sem).wait()
        pltpu.make_async_copy(pys.at[idc.at[j]], pc.at[b, 1], sem).wait()
        pltpu.make_async_copy(pzs.at[idc.at[j]], pc.at[b, 2], sem).wait()
        for k in range(CHUNK // LANES):
            sl = pl.ds(k * LANES, LANES)
            bufs[0, sl] = pr[b, 0, sl] - pc[b, 0, sl]
            bufs[1, sl] = pr[b, 1, sl] - pc[b, 1, sl]
            bufs[2, sl] = pr[b, 2, sl] - pc[b, 2, sl]
        pltpu.sync_copy(bufu.at[b], r2_out.at[pl.ds(base, CHUNK)])
        pltpu.sync_copy(bufs, ss_out.at[:, pl.ds(base, CHUNK)])

    issue(0, 0)

    def body(jj, carry):
        j = 2 * jj
        issue(j + 1, 1)
        drain(j, 0)
        issue(j + 2, 0)
        drain(j + 1, 1)
        return carry

    lax.fori_loop(0, nch // 2 - 1, body, 0)
    issue(nch - 1, 1)
    drain(nch - 2, 0)
    drain(nch - 1, 1)


def _mlp_body(h_dim, r2, ss, w1s, w2, b2, w3e, b3e, out):
    f1 = r2[...][:, 0:h_dim]
    ssv = ss[...]
    dx = ssv[0:1, :]
    dy = ssv[1:2, :]
    dz = ssv[2:3, :]
    n2 = dx * dx + dy * dy + dz * dz
    inv = lax.rsqrt(jnp.maximum(n2, 1e-16))
    x = dx * inv
    y = dy * inv
    z = dz * inv
    p1 = x * x - y * y
    p2 = z * x
    p3 = 2.0 * z * z - x * x - y * y
    ones = jnp.ones_like(x)
    st = jnp.concatenate([x, z, p1, p2, p3, ones], axis=0)
    w1sv = w1s[...]
    c0 = 0.5 / math.sqrt(math.pi)
    cx = 0.5 * math.sqrt(3.0 / (2.0 * math.pi))
    cz = 0.5 * math.sqrt(3.0 / math.pi)
    cxy = 0.25 * math.sqrt(15.0 / (2.0 * math.pi))
    czx = 0.5 * math.sqrt(15.0 / (2.0 * math.pi))
    czz = 0.25 * math.sqrt(5.0 / math.pi)
    us = jnp.concatenate([
        cx * (w1sv[3:4, :] - w1sv[1:2, :]),
        cz * w1sv[2:3, :],
        cxy * (w1sv[4:5, :] + w1sv[8:9, :]),
        czx * (w1sv[7:8, :] - w1sv[5:6, :]),
        czz * w1sv[6:7, :],
        c0 * w1sv[0:1, :],
    ], axis=0)
    hsh = lax.dot_general(st, us, (((0,), (0,)), ((), ())),
                          preferred_element_type=jnp.float32)
    h = f1 + hsh
    h = h * jax.nn.sigmoid(h)
    h = jnp.dot(h, w2[...], preferred_element_type=jnp.float32) + b2[...]
    h = h * jax.nn.sigmoid(h)
    out[...] = jnp.dot(h, w3e[...], preferred_element_type=jnp.float32) \
        + b3e[...]


def _scatter_body(nch_tot, off, nch, rt, rows_hbm, msg_hbm, zeros_hbm,
                  part_out, idr, mbuf, accum, semr0, semr1):
    cid = lax.axis_index("c")
    sid = lax.axis_index("s")
    wid = sid * NC + cid
    pltpu.sync_copy(zeros_hbm.at[pl.ds(sid * rt, rt)],
                    accum.at[pl.ds(sid * rt, rt)])
    pltpu.sync_copy(rows_hbm.at[pl.ds(wid * nch_tot + off, nch)], idr)
    plsc.subcore_barrier()
    sems = [semr0, semr1]

    def rd(j, b):
        base = (wid * nch + j) * CHUNK
        pltpu.async_copy(msg_hbm.at[pl.ds(base, CHUNK)], mbuf.at[b], sems[b])

    def sc(j, b):
        base = (wid * nch + j) * CHUNK
        pltpu.make_async_copy(msg_hbm.at[pl.ds(base, CHUNK)], mbuf.at[b],
                              sems[b]).wait()
        pltpu.sync_copy(mbuf.at[b], accum.at[idr.at[j]], add=True)

    rd(0, 0)

    def body(jj, carry):
        j = 2 * jj
        rd(j + 1, 1)
        sc(j, 0)
        rd(j + 2, 0)
        sc(j + 1, 1)
        return carry

    lax.fori_loop(0, nch // 2 - 1, body, 0)
    rd(nch - 1, 1)
    sc(nch - 2, 0)
    sc(nch - 1, 1)
    plsc.subcore_barrier()
    pltpu.sync_copy(accum.at[pl.ds(sid * rt, rt)],
                    part_out.at[cid, pl.ds(sid * rt, rt)])


def _combine_body(p0, p1, out):
    a = p0[...]
    b = p1[...]
    out[...] = (a[0] + a[1]) + (b[0] + b[1])


def kernel(features, positions, edge_index, W1, b1, W2, b2, W3, b3):
    n, c_in = features.shape
    e = edge_index.shape[1]
    n_sh = 9
    h_dim = W1.shape[1]
    c_out = W3.shape[1] // n_sh
    f32 = jnp.float32

    ei = edge_index.astype(jnp.int32)
    row = ei[0]
    col = ei[1]
    epad = -(-e // (NW * CHUNK)) * (NW * CHUNK)
    nch = epad // (NW * CHUNK)
    pad = epad - e
    rowg = jnp.concatenate([row, jnp.zeros((pad,), jnp.int32)])
    rows = jnp.concatenate([row, jnp.full((pad,), n, jnp.int32)])
    colg = jnp.concatenate([col, jnp.zeros((pad,), jnp.int32)])
    rowg = rowg.reshape(-1, CHUNK)
    rows = rows.reshape(-1, CHUNK)
    colg = colg.reshape(-1, CHUNK)

    npad = -(-(n + 1) // (NS * 8)) * (NS * 8)
    rt = npad // NS
    n8 = -(-n // 8) * 8
    posp = jnp.concatenate(
        [positions.astype(f32), jnp.zeros((n8 - n, 3), f32)], axis=0)
    px = posp[:, 0]
    py = posp[:, 1]
    pz = posp[:, 2]

    w1n = W1[n_sh:]
    w1s = jnp.concatenate([W1[:n_sh], jnp.zeros((16 - n_sh, h_dim), f32)])
    b1r = b1.reshape(1, h_dim)
    b2r = b2.reshape(1, h_dim)
    b3r = b3.reshape(1, n_sh * c_out)

    nb = n // BN
    u_tab, w3e, b3e = pl.pallas_call(
        functools.partial(_prep_body, n_sh, c_out),
        grid=(nb,),
        in_specs=[
            pl.BlockSpec((BN, c_in), lambda i: (i, 0)),
            pl.BlockSpec((c_in, h_dim), lambda i: (0, 0)),
            pl.BlockSpec((1, h_dim), lambda i: (0, 0)),
            pl.BlockSpec((h_dim, n_sh * c_out), lambda i: (0, 0)),
            pl.BlockSpec((1, n_sh * c_out), lambda i: (0, 0)),
        ],
        out_specs=[
            pl.BlockSpec((BN, 128), lambda i: (i, 0)),
            pl.BlockSpec((h_dim, c_out), lambda i: (0, 0)),
            pl.BlockSpec((1, c_out), lambda i: (0, 0)),
        ],
        out_shape=[
            jax.ShapeDtypeStruct((n, 128), f32),
            jax.ShapeDtypeStruct((h_dim, c_out), f32),
            jax.ShapeDtypeStruct((1, c_out), f32),
        ],
    )(features, w1n, b1r, W3, b3r)

    mesh = plsc.VectorSubcoreMesh(core_axis_name="c", subcore_axis_name="s")
    zeros_init = jnp.zeros((npad, c_out), f32)
    nchh0 = (nch * 3 // 5) // 8 * 8
    splits = [(0, nchh0), (nchh0, nch - nchh0)]

    def stage_b(off, nchh):
        eph = nchh * NW * CHUNK
        return pl.kernel(
            functools.partial(_gather_body, nch, off, nchh, h_dim),
            out_type=[
                jax.ShapeDtypeStruct((eph, 128), f32),
                jax.ShapeDtypeStruct((8, eph), f32),
            ],
            mesh=mesh,
            scratch_types=[
                pltpu.VMEM((nchh, CHUNK), jnp.int32),
                pltpu.VMEM((nchh, CHUNK), jnp.int32),
                pltpu.VMEM((2, 3, CHUNK), f32),
                pltpu.VMEM((2, 3, CHUNK), f32),
                pltpu.VMEM((2, CHUNK, 128), f32),
                pltpu.VMEM((8, CHUNK), f32),
                pltpu.VMEM_SHARED((n, 128), f32),
                pltpu.VMEM_SHARED((n8,), f32),
                pltpu.VMEM_SHARED((n8,), f32),
                pltpu.VMEM_SHARED((n8,), f32),
                pltpu.SemaphoreType.DMA,
                pltpu.SemaphoreType.DMA,
            ],
        )(rowg, colg, u_tab, px, py, pz)

    def stage_c(r2t, ss):
        eph = r2t.shape[0]
        return pl.pallas_call(
            functools.partial(_mlp_body, h_dim),
            grid=(eph // BE,),
            in_specs=[
                pl.BlockSpec((BE, 128), lambda i: (i, 0)),
                pl.BlockSpec((8, BE), lambda i: (0, i)),
                pl.BlockSpec((16, h_dim), lambda i: (0, 0)),
                pl.BlockSpec((h_dim, h_dim), lambda i: (0, 0)),
                pl.BlockSpec((1, h_dim), lambda i: (0, 0)),
                pl.BlockSpec((h_dim, c_out), lambda i: (0, 0)),
                pl.BlockSpec((1, c_out), lambda i: (0, 0)),
            ],
            out_specs=pl.BlockSpec((BE, c_out), lambda i: (i, 0)),
            out_shape=jax.ShapeDtypeStruct((eph, c_out), f32),
        )(r2t, ss, w1s, W2, b2r, w3e, b3e)

    def stage_d(off, nchh, msg_h):
        return pl.kernel(
            functools.partial(_scatter_body, nch, off, nchh, rt),
            out_type=jax.ShapeDtypeStruct((NC, npad, c_out), f32),
            mesh=mesh,
            scratch_types=[
                pltpu.VMEM((nchh, CHUNK), jnp.int32),
                pltpu.VMEM((2, CHUNK, c_out), f32),
                pltpu.VMEM_SHARED((npad, c_out), f32),
                pltpu.SemaphoreType.DMA,
                pltpu.SemaphoreType.DMA,
            ],
        )(rows, msg_h, zeros_init)

    r2t0, ss0 = stage_b(*splits[0])
    r2t1, ss1 = stage_b(*splits[1])
    msg0 = stage_c(r2t0, ss0)
    msg1 = stage_c(r2t1, ss1)
    part0 = stage_d(splits[0][0], splits[0][1], msg0)
    part1 = stage_d(splits[1][0], splits[1][1], msg1)

    out = pl.pallas_call(
        _combine_body,
        grid=(nb,),
        in_specs=[
            pl.BlockSpec((NC, BN, c_out), lambda i: (0, i, 0)),
            pl.BlockSpec((NC, BN, c_out), lambda i: (0, i, 0)),
        ],
        out_specs=pl.BlockSpec((BN, c_out), lambda i: (i, 0)),
        out_shape=jax.ShapeDtypeStruct((n, c_out), f32),
    )(part0, part1)
    return out
